# Initial kernel scaffold; baseline (speedup 1.0000x reference)
import jax
import jax.numpy as jnp
from jax.experimental import pallas as pl


def kernel(image, question, table):
    raise NotImplementedError("write your pallas kernel here")


# speedup vs baseline: 1.4610x; 1.4610x over previous
import functools

import jax
import jax.numpy as jnp
from jax import lax
from jax.experimental import pallas as pl
from jax.experimental.pallas import tpu as pltpu
from jax.experimental.pallas import tpu_sc as plsc

VOCAB = 1000000
EMBED_DIM = 32
BATCH = 16384
SEQ = 20

NUM_CORES = 2
NUM_SUBCORES = 16
NUM_WORKERS = NUM_CORES * NUM_SUBCORES

B_TOTAL = BATCH * SEQ
B_PER_W = B_TOTAL // NUM_WORKERS
CHUNK = 1024
N_CHUNKS = B_PER_W // CHUNK


def _gather_body(idx_hbm, table_hbm, out_hbm, idx_v, rows_a, rows_b,
                 gsem_a, gsem_b, wsem):
    wid = lax.axis_index("s") * NUM_CORES + lax.axis_index("c")
    base = wid * B_PER_W
    pltpu.sync_copy(idx_hbm.at[pl.ds(base, B_PER_W)], idx_v)

    bufs = (rows_a, rows_b)
    gsems = (gsem_a, gsem_b)

    pltpu.async_copy(table_hbm.at[idx_v.at[pl.ds(0, CHUNK)]], rows_a, gsem_a)
    for c in range(N_CHUNKS):
        cur = bufs[c % 2]
        if c + 1 < N_CHUNKS:
            nxt = bufs[(c + 1) % 2]
            if c >= 1:
                pltpu.make_async_copy(
                    nxt, out_hbm.at[pl.ds(base + (c - 1) * CHUNK, CHUNK)],
                    wsem).wait()
            pltpu.async_copy(
                table_hbm.at[idx_v.at[pl.ds((c + 1) * CHUNK, CHUNK)]],
                nxt, gsems[(c + 1) % 2])
        pltpu.make_async_copy(
            table_hbm.at[idx_v.at[pl.ds(c * CHUNK, CHUNK)]], cur,
            gsems[c % 2]).wait()
        pltpu.async_copy(cur, out_hbm.at[pl.ds(base + c * CHUNK, CHUNK)], wsem)
    pltpu.make_async_copy(
        bufs[(N_CHUNKS - 2) % 2],
        out_hbm.at[pl.ds(base + (N_CHUNKS - 2) * CHUNK, CHUNK)], wsem).wait()
    pltpu.make_async_copy(
        bufs[(N_CHUNKS - 1) % 2],
        out_hbm.at[pl.ds(base + (N_CHUNKS - 1) * CHUNK, CHUNK)], wsem).wait()


@jax.jit
def _sc_gather(idx_flat, table):
    mesh = plsc.VectorSubcoreMesh(core_axis_name="c", subcore_axis_name="s")
    return pl.kernel(
        _gather_body,
        out_type=jax.ShapeDtypeStruct((B_TOTAL, EMBED_DIM), jnp.float32),
        mesh=mesh,
        scratch_types=[
            pltpu.VMEM((B_PER_W,), jnp.int32),
            pltpu.VMEM((CHUNK, EMBED_DIM), jnp.float32),
            pltpu.VMEM((CHUNK, EMBED_DIM), jnp.float32),
            pltpu.SemaphoreType.DMA,
            pltpu.SemaphoreType.DMA,
            pltpu.SemaphoreType.DMA,
        ],
        compiler_params=pltpu.CompilerParams(use_tc_tiling_on_sc=False),
    )(idx_flat, table)


def kernel(image, question, table):
    idx_flat = question.reshape(-1).astype(jnp.int32)
    emb = _sc_gather(idx_flat, table)
    return (image, emb.reshape(BATCH, SEQ, EMBED_DIM))


# speedup vs baseline: 1.6643x; 1.1392x over previous
import functools

import jax
import jax.numpy as jnp
from jax import lax
from jax.experimental import pallas as pl
from jax.experimental.pallas import tpu as pltpu
from jax.experimental.pallas import tpu_sc as plsc

VOCAB = 1000000
EMBED_DIM = 32
BATCH = 16384
SEQ = 20

NUM_CORES = 2
NUM_SUBCORES = 16
NUM_WORKERS = NUM_CORES * NUM_SUBCORES

B_TOTAL = BATCH * SEQ
B_PER_W = B_TOTAL // NUM_WORKERS
R_PER_W = BATCH // NUM_WORKERS
RB = 64
CHUNK = RB * SEQ
N_CHUNKS = B_PER_W // CHUNK


def _gather_body(idx_hbm, table_hbm, out_hbm, idx_v, rows_a, rows_b,
                 gsem_a, gsem_b, wsem):
    wid = lax.axis_index("s") * NUM_CORES + lax.axis_index("c")
    base = wid * B_PER_W
    brow0 = wid * R_PER_W
    pltpu.sync_copy(idx_hbm.at[pl.ds(base, B_PER_W)], idx_v)

    bufs = (rows_a, rows_b)
    gsems = (gsem_a, gsem_b)

    def start_writes(buf, c):
        b0 = brow0 + c * RB
        for k in range(RB):
            pltpu.async_copy(buf.at[pl.ds(k * SEQ, SEQ)], out_hbm.at[b0 + k],
                             wsem)

    def drain_writes():
        pltpu.make_async_copy(table_hbm.at[pl.ds(0, CHUNK)], rows_a,
                              wsem).wait()

    pltpu.async_copy(table_hbm.at[idx_v.at[pl.ds(0, CHUNK)]], rows_a, gsem_a)
    for c in range(N_CHUNKS):
        cur = bufs[c % 2]
        if c + 1 < N_CHUNKS:
            if c >= 1:
                drain_writes()
            pltpu.async_copy(
                table_hbm.at[idx_v.at[pl.ds((c + 1) * CHUNK, CHUNK)]],
                bufs[(c + 1) % 2], gsems[(c + 1) % 2])
        pltpu.make_async_copy(
            table_hbm.at[idx_v.at[pl.ds(c * CHUNK, CHUNK)]], cur,
            gsems[c % 2]).wait()
        start_writes(cur, c)
    drain_writes()
    drain_writes()


FMT_COLS = 32768
FMT_GRID = -(-VOCAB // FMT_COLS)


def _fmt_body(in_ref, out_ref):
    x = in_ref[...]
    y = jnp.transpose(x, (1, 0))
    y3 = y.reshape(FMT_COLS // 4, 4, EMBED_DIM)
    parts = [y3[:, q, :] for q in range(4)]
    out_ref[...] = jnp.concatenate(parts, axis=1)


@jax.jit
def _tc_format(table_t):
    return pl.pallas_call(
        _fmt_body,
        grid=(FMT_GRID,),
        in_specs=[pl.BlockSpec((EMBED_DIM, FMT_COLS), lambda i: (0, i))],
        out_specs=pl.BlockSpec((FMT_COLS // 4, 128), lambda i: (i, 0)),
        out_shape=jax.ShapeDtypeStruct((VOCAB // 4, 128), jnp.float32),
    )(table_t)


@jax.jit
def _sc_gather(idx_flat, table):
    mesh = plsc.VectorSubcoreMesh(core_axis_name="c", subcore_axis_name="s")
    return pl.kernel(
        _gather_body,
        out_type=jax.ShapeDtypeStruct((BATCH, SEQ, EMBED_DIM), jnp.float32),
        mesh=mesh,
        scratch_types=[
            pltpu.VMEM((B_PER_W,), jnp.int32),
            pltpu.VMEM((CHUNK, EMBED_DIM), jnp.float32),
            pltpu.VMEM((CHUNK, EMBED_DIM), jnp.float32),
            pltpu.SemaphoreType.DMA,
            pltpu.SemaphoreType.DMA,
            pltpu.SemaphoreType.DMA,
        ],
        compiler_params=pltpu.CompilerParams(use_tc_tiling_on_sc=False),
    )(idx_flat, table)


def kernel(image, question, table):
    idx_flat = question.reshape(-1).astype(jnp.int32)
    tbl_lin = _tc_format(table.swapaxes(0, 1))
    emb = _sc_gather(idx_flat, tbl_lin.reshape(VOCAB, EMBED_DIM))
    return (image, emb)
